# baseline (speedup 1.0000x reference)
import functools

import jax
import jax.numpy as jnp
from jax.experimental import pallas as pl
from jax.experimental.pallas import tpu as pltpu
from jax.experimental.pallas import tpu_sc as plsc

B, T, D = 4, 1024, 1024
H = 512
LAT = 1024
K = 8192
CD = 8

_f32 = jnp.float32
_bf16 = jnp.bfloat16

_NW = 32
_TPW = (B * T) // _NW
_NGRP = _TPW // 16
_GPP = 4


def _bdot(a_bf, b_bf):
    return jnp.dot(a_bf, b_bf, preferred_element_type=_f32)


def _enc_body(x_ref, w1_ref, b1_ref, w2_ref, b2_ref, w3_ref, b3_ref,
              pw_ref, pb_ref, zn_ref):
    x = x_ref[0]

    def conv3tap(inp_bf, w_ref, b_row, width):
        y0 = _bdot(inp_bf, w_ref[0])
        y1 = _bdot(inp_bf, w_ref[1])
        y2 = _bdot(inp_bf, w_ref[2])
        zrow = jnp.zeros((1, width), _f32)
        return (jnp.concatenate([zrow, y0[:-1]], axis=0) + y1
                + jnp.concatenate([y2[1:], zrow], axis=0) + b_row)

    h1 = jax.nn.gelu(conv3tap(x, w1_ref, b1_ref[...], H))
    h2 = jax.nn.gelu(conv3tap(h1.astype(_bf16), w2_ref, b2_ref[...], H) + h1)
    z = _bdot(h2.astype(_bf16), w3_ref[...]) + b3_ref[...]
    zp = _bdot(z.astype(_bf16), pw_ref[...]) + pb_ref[...]
    zn = zp / (jnp.sqrt(jnp.sum(zp * zp, axis=1, keepdims=True)) + 1e-8)
    zn_ref[0] = zn.astype(_bf16)


def _tok_body(zt_hbm, cb_hbm, out_hbm, cb_v, z_v, tok_v):
    cid = jax.lax.axis_index("c")
    sid = jax.lax.axis_index("s")
    wid = sid * 2 + cid
    base = wid * _TPW
    pltpu.sync_copy(cb_hbm, cb_v)
    pltpu.sync_copy(zt_hbm.at[:, pl.ds(base, _TPW)], z_v)
    ninf = jnp.full((16,), -jnp.inf, _f32)
    zero = jnp.zeros((16,), jnp.int32)
    for g0 in range(0, _NGRP, _GPP):
        zs = [[z_v[c, pl.ds((g0 + j) * 16, 16)] for c in range(CD)]
              for j in range(_GPP)]

        def body(k0, carry, zs=zs):
            kbase = k0 * 16
            cvecs = [cb_v[c, pl.ds(kbase, 16)] for c in range(CD)]
            kvec0 = jnp.full((16,), kbase, jnp.int32)
            out = list(carry)
            for t in range(16):
                s = [cvecs[c][t] for c in range(CD)]
                kvec = kvec0 + t
                for j in range(_GPP):
                    bv, bi = out[2 * j], out[2 * j + 1]
                    sim = zs[j][0] * s[0]
                    for c in range(1, CD):
                        sim = sim + zs[j][c] * s[c]
                    m = sim > bv
                    out[2 * j] = jnp.where(m, sim, bv)
                    out[2 * j + 1] = jnp.where(m, kvec, bi)
            return tuple(out)

        carry = jax.lax.fori_loop(0, K // 16, body, (ninf, zero) * _GPP)
        for j in range(_GPP):
            tok_v[pl.ds((g0 + j) * 16, 16)] = carry[2 * j + 1]
    pltpu.sync_copy(tok_v, out_hbm.at[pl.ds(base, _TPW)])


def kernel(features, w1, b1, w2, b2, w3, b3, proj_w, proj_b, codebook):
    x_bf = features.astype(_bf16)
    w1k = jnp.transpose(w1, (2, 1, 0)).astype(_bf16)
    w2k = jnp.transpose(w2, (2, 1, 0)).astype(_bf16)
    w3t = jnp.transpose(w3[:, :, 0], (1, 0)).astype(_bf16)
    pwt = jnp.transpose(proj_w, (1, 0)).astype(_bf16)
    cbn = codebook / (jnp.linalg.norm(codebook, axis=-1, keepdims=True) + 1e-8)
    cb_bf = jax.lax.optimization_barrier(jnp.transpose(cbn, (1, 0)).astype(_bf16))
    cbf = cb_bf.astype(_f32)
    b1r = b1.reshape(1, H)
    b2r = b2.reshape(1, H)
    b3r = b3.reshape(1, LAT)
    pbr = proj_b.reshape(1, CD)
    zn = pl.pallas_call(
        _enc_body,
        grid=(B,),
        in_specs=[
            pl.BlockSpec((1, T, D), lambda b: (b, 0, 0)),
            pl.BlockSpec((3, D, H), lambda b: (0, 0, 0)),
            pl.BlockSpec((1, H), lambda b: (0, 0)),
            pl.BlockSpec((3, H, H), lambda b: (0, 0, 0)),
            pl.BlockSpec((1, H), lambda b: (0, 0)),
            pl.BlockSpec((H, LAT), lambda b: (0, 0)),
            pl.BlockSpec((1, LAT), lambda b: (0, 0)),
            pl.BlockSpec((LAT, CD), lambda b: (0, 0)),
            pl.BlockSpec((1, CD), lambda b: (0, 0)),
        ],
        out_specs=pl.BlockSpec((1, T, CD), lambda b: (b, 0, 0)),
        out_shape=jax.ShapeDtypeStruct((B, T, CD), _bf16),
    )(x_bf, w1k, b1r, w2k, b2r, w3t, b3r, pwt, pbr)
    zt = jnp.transpose(zn.reshape(B * T, CD), (1, 0)).astype(_f32)
    tok = pl.kernel(
        _tok_body,
        out_type=jax.ShapeDtypeStruct((B * T,), jnp.int32),
        mesh=plsc.VectorSubcoreMesh(core_axis_name="c", subcore_axis_name="s"),
        scratch_types=[
            pltpu.VMEM((CD, K), _f32),
            pltpu.VMEM((CD, _TPW), _f32),
            pltpu.VMEM((_TPW,), jnp.int32),
        ],
    )(zt, cbf)
    return tok.reshape(B, T)


# speedup vs baseline: 2.0410x; 2.0410x over previous
import functools

import jax
import jax.numpy as jnp
from jax.experimental import pallas as pl
from jax.experimental.pallas import tpu as pltpu
from jax.experimental.pallas import tpu_sc as plsc

B, T, D = 4, 1024, 1024
H = 512
LAT = 1024
K = 8192
CD = 8

K_SC = 1536
KC = 512

_f32 = jnp.float32
_bf16 = jnp.bfloat16

_NW = 32
_TPW = (B * T) // _NW
_NGRP = _TPW // 16
_GPP = 4


def _bdot(a_bf, b_bf):
    return jnp.dot(a_bf, b_bf, preferred_element_type=_f32)


def _enc_body(x_ref, w1_ref, b1_ref, w2_ref, b2_ref, w3_ref, b3_ref,
              pw_ref, pb_ref, cbn_ref, zn_ref, bv_ref, bi_ref):
    x = x_ref[0]

    def conv3tap(inp_bf, w_ref, b_row, width):
        y0 = _bdot(inp_bf, w_ref[0])
        y1 = _bdot(inp_bf, w_ref[1])
        y2 = _bdot(inp_bf, w_ref[2])
        zrow = jnp.zeros((1, width), _f32)
        return (jnp.concatenate([zrow, y0[:-1]], axis=0) + y1
                + jnp.concatenate([y2[1:], zrow], axis=0) + b_row)

    h1 = jax.nn.gelu(conv3tap(x, w1_ref, b1_ref[...], H))
    h2 = jax.nn.gelu(conv3tap(h1.astype(_bf16), w2_ref, b2_ref[...], H) + h1)
    z = _bdot(h2.astype(_bf16), w3_ref[...]) + b3_ref[...]
    zp = _bdot(z.astype(_bf16), pw_ref[...]) + pb_ref[...]
    zn = zp / (jnp.sqrt(jnp.sum(zp * zp, axis=1, keepdims=True)) + 1e-8)
    zn_bf = zn.astype(_bf16)
    zn_ref[0] = zn_bf

    best_v = jnp.full((T, 1), -jnp.inf, _f32)
    best_i = jnp.zeros((T, 1), jnp.int32)
    for kc in range(0, K - K_SC, KC):
        s = _bdot(zn_bf, cbn_ref[:, kc:kc + KC])
        m = jnp.max(s, axis=1, keepdims=True)
        idx = jax.lax.broadcasted_iota(jnp.int32, s.shape, 1) + (K_SC + kc)
        cand = jnp.min(jnp.where(s == m, idx, K), axis=1, keepdims=True)
        upd = m > best_v
        best_v = jnp.where(upd, m, best_v)
        best_i = jnp.where(upd, cand, best_i)
    bv_ref[0] = best_v
    bi_ref[0] = best_i


def _tok_body(zt_hbm, cb_hbm, bv_hbm, bi_hbm, cb_v, z_v, val_v, idx_v):
    cid = jax.lax.axis_index("c")
    sid = jax.lax.axis_index("s")
    wid = sid * 2 + cid
    base = wid * _TPW
    pltpu.sync_copy(cb_hbm, cb_v)
    pltpu.sync_copy(zt_hbm.at[:, pl.ds(base, _TPW)], z_v)
    ninf = jnp.full((16,), -jnp.inf, _f32)
    zero = jnp.zeros((16,), jnp.int32)
    for g0 in range(0, _NGRP, _GPP):
        zs = [[z_v[c, pl.ds((g0 + j) * 16, 16)] for c in range(CD)]
              for j in range(_GPP)]

        def body(k0, carry, zs=zs):
            kbase = k0 * 16
            cvecs = [cb_v[c, pl.ds(kbase, 16)] for c in range(CD)]
            kvec0 = jnp.full((16,), kbase, jnp.int32)
            out = list(carry)
            for t in range(16):
                s = [cvecs[c][t] for c in range(CD)]
                kvec = kvec0 + t
                for j in range(_GPP):
                    bv, bi = out[2 * j], out[2 * j + 1]
                    sim = zs[j][0] * s[0]
                    for c in range(1, CD):
                        sim = sim + zs[j][c] * s[c]
                    m = sim > bv
                    out[2 * j] = jnp.where(m, sim, bv)
                    out[2 * j + 1] = jnp.where(m, kvec, bi)
            return tuple(out)

        carry = jax.lax.fori_loop(0, K_SC // 16, body, (ninf, zero) * _GPP)
        for j in range(_GPP):
            val_v[pl.ds((g0 + j) * 16, 16)] = carry[2 * j]
            idx_v[pl.ds((g0 + j) * 16, 16)] = carry[2 * j + 1]
    pltpu.sync_copy(val_v, bv_hbm.at[pl.ds(base, _TPW)])
    pltpu.sync_copy(idx_v, bi_hbm.at[pl.ds(base, _TPW)])


def kernel(features, w1, b1, w2, b2, w3, b3, proj_w, proj_b, codebook):
    x_bf = features.astype(_bf16)
    w1k = jnp.transpose(w1, (2, 1, 0)).astype(_bf16)
    w2k = jnp.transpose(w2, (2, 1, 0)).astype(_bf16)
    w3t = jnp.transpose(w3[:, :, 0], (1, 0)).astype(_bf16)
    pwt = jnp.transpose(proj_w, (1, 0)).astype(_bf16)
    cbn = codebook / (jnp.linalg.norm(codebook, axis=-1, keepdims=True) + 1e-8)
    cb_bf = jnp.transpose(cbn, (1, 0)).astype(_bf16)
    cb_sc = jax.lax.optimization_barrier(cb_bf[:, :K_SC]).astype(_f32)
    cb_tc = cb_bf[:, K_SC:]
    b1r = b1.reshape(1, H)
    b2r = b2.reshape(1, H)
    b3r = b3.reshape(1, LAT)
    pbr = proj_b.reshape(1, CD)
    zn, tc_v, tc_i = pl.pallas_call(
        _enc_body,
        grid=(B,),
        in_specs=[
            pl.BlockSpec((1, T, D), lambda b: (b, 0, 0)),
            pl.BlockSpec((3, D, H), lambda b: (0, 0, 0)),
            pl.BlockSpec((1, H), lambda b: (0, 0)),
            pl.BlockSpec((3, H, H), lambda b: (0, 0, 0)),
            pl.BlockSpec((1, H), lambda b: (0, 0)),
            pl.BlockSpec((H, LAT), lambda b: (0, 0)),
            pl.BlockSpec((1, LAT), lambda b: (0, 0)),
            pl.BlockSpec((LAT, CD), lambda b: (0, 0)),
            pl.BlockSpec((1, CD), lambda b: (0, 0)),
            pl.BlockSpec((CD, K - K_SC), lambda b: (0, 0)),
        ],
        out_specs=[
            pl.BlockSpec((1, T, CD), lambda b: (b, 0, 0)),
            pl.BlockSpec((1, T, 1), lambda b: (b, 0, 0)),
            pl.BlockSpec((1, T, 1), lambda b: (b, 0, 0)),
        ],
        out_shape=[
            jax.ShapeDtypeStruct((B, T, CD), _bf16),
            jax.ShapeDtypeStruct((B, T, 1), _f32),
            jax.ShapeDtypeStruct((B, T, 1), jnp.int32),
        ],
    )(x_bf, w1k, b1r, w2k, b2r, w3t, b3r, pwt, pbr, cb_tc)
    zt = jnp.transpose(zn.reshape(B * T, CD), (1, 0)).astype(_f32)
    sc_v, sc_i = pl.kernel(
        _tok_body,
        out_type=[
            jax.ShapeDtypeStruct((B * T,), _f32),
            jax.ShapeDtypeStruct((B * T,), jnp.int32),
        ],
        mesh=plsc.VectorSubcoreMesh(core_axis_name="c", subcore_axis_name="s"),
        scratch_types=[
            pltpu.VMEM((CD, K_SC), _f32),
            pltpu.VMEM((CD, _TPW), _f32),
            pltpu.VMEM((_TPW,), _f32),
            pltpu.VMEM((_TPW,), jnp.int32),
        ],
    )(zt, cb_sc)
    tc_vf = tc_v.reshape(B * T)
    tc_if = tc_i.reshape(B * T)
    tok = jnp.where(sc_v >= tc_vf, sc_i, tc_if)
    return tok.reshape(B, T)


# speedup vs baseline: 2.5850x; 1.2665x over previous
import functools

import jax
import jax.numpy as jnp
from jax.experimental import pallas as pl
from jax.experimental.pallas import tpu as pltpu
from jax.experimental.pallas import tpu_sc as plsc

B, T, D = 4, 1024, 1024
H = 512
LAT = 1024
K = 8192
CD = 8

K_SC = 1536
KC = 512

_f32 = jnp.float32
_bf16 = jnp.bfloat16

_NW = 32
_TPW = (B * T) // _NW
_NGRP = _TPW // 16
_GPP = 4


def _bdot(a_bf, b_bf):
    return jnp.dot(a_bf, b_bf, preferred_element_type=_f32)


def _enc_body(x_ref, w1_ref, b1_ref, w2_ref, b2_ref, w3_ref, b3_ref,
              pw_ref, pb_ref, zn_ref):
    x = x_ref[0]

    def conv3tap(inp_bf, w_ref, b_row, width):
        y0 = _bdot(inp_bf, w_ref[0])
        y1 = _bdot(inp_bf, w_ref[1])
        y2 = _bdot(inp_bf, w_ref[2])
        zrow = jnp.zeros((1, width), _f32)
        return (jnp.concatenate([zrow, y0[:-1]], axis=0) + y1
                + jnp.concatenate([y2[1:], zrow], axis=0) + b_row)

    h1 = jax.nn.gelu(conv3tap(x, w1_ref, b1_ref[...], H))
    h2 = jax.nn.gelu(conv3tap(h1.astype(_bf16), w2_ref, b2_ref[...], H) + h1)
    z = _bdot(h2.astype(_bf16), w3_ref[...]) + b3_ref[...]
    zp = _bdot(z.astype(_bf16), pw_ref[...]) + pb_ref[...]
    zn = zp / (jnp.sqrt(jnp.sum(zp * zp, axis=1, keepdims=True)) + 1e-8)
    zn_ref[0] = zn.astype(_bf16)


def _scan_tc_body(zn_ref, cbn_ref, bv_ref, bi_ref):
    zn_bf = zn_ref[0]
    best_v = jnp.full((T, 1), -jnp.inf, _f32)
    best_i = jnp.zeros((T, 1), jnp.int32)
    for kc in range(0, K - K_SC, KC):
        s = _bdot(zn_bf, cbn_ref[:, kc:kc + KC])
        m = jnp.max(s, axis=1, keepdims=True)
        idx = jax.lax.broadcasted_iota(jnp.int32, s.shape, 1) + (K_SC + kc)
        cand = jnp.min(jnp.where(s == m, idx, K), axis=1, keepdims=True)
        upd = m > best_v
        best_v = jnp.where(upd, m, best_v)
        best_i = jnp.where(upd, cand, best_i)
    bv_ref[0] = best_v
    bi_ref[0] = best_i


def _tok_body(zt_hbm, cb_hbm, bv_hbm, bi_hbm, cb_v, z_v, val_v, idx_v):
    cid = jax.lax.axis_index("c")
    sid = jax.lax.axis_index("s")
    wid = sid * 2 + cid
    base = wid * _TPW
    pltpu.sync_copy(cb_hbm, cb_v)
    pltpu.sync_copy(zt_hbm.at[:, pl.ds(base, _TPW)], z_v)
    ninf = jnp.full((16,), -jnp.inf, _f32)
    zero = jnp.zeros((16,), jnp.int32)
    for g0 in range(0, _NGRP, _GPP):
        zs = [[z_v[c, pl.ds((g0 + j) * 16, 16)] for c in range(CD)]
              for j in range(_GPP)]

        def body(k0, carry, zs=zs):
            kbase = k0 * 16
            cvecs = [cb_v[c, pl.ds(kbase, 16)] for c in range(CD)]
            kvec0 = jnp.full((16,), kbase, jnp.int32)
            out = list(carry)
            for t in range(16):
                s = [cvecs[c][t] for c in range(CD)]
                kvec = kvec0 + t
                for j in range(_GPP):
                    bv, bi = out[2 * j], out[2 * j + 1]
                    sim = zs[j][0] * s[0]
                    for c in range(1, CD):
                        sim = sim + zs[j][c] * s[c]
                    m = sim > bv
                    out[2 * j] = jnp.where(m, sim, bv)
                    out[2 * j + 1] = jnp.where(m, kvec, bi)
            return tuple(out)

        carry = jax.lax.fori_loop(0, K_SC // 16, body, (ninf, zero) * _GPP)
        for j in range(_GPP):
            val_v[pl.ds((g0 + j) * 16, 16)] = carry[2 * j]
            idx_v[pl.ds((g0 + j) * 16, 16)] = carry[2 * j + 1]
    pltpu.sync_copy(val_v, bv_hbm.at[pl.ds(base, _TPW)])
    pltpu.sync_copy(idx_v, bi_hbm.at[pl.ds(base, _TPW)])


def kernel(features, w1, b1, w2, b2, w3, b3, proj_w, proj_b, codebook):
    x_bf = features.astype(_bf16)
    w1k = jnp.transpose(w1, (2, 1, 0)).astype(_bf16)
    w2k = jnp.transpose(w2, (2, 1, 0)).astype(_bf16)
    w3t = jnp.transpose(w3[:, :, 0], (1, 0)).astype(_bf16)
    pwt = jnp.transpose(proj_w, (1, 0)).astype(_bf16)
    cbn = codebook / (jnp.linalg.norm(codebook, axis=-1, keepdims=True) + 1e-8)
    cb_bf = jnp.transpose(cbn, (1, 0)).astype(_bf16)
    cb_sc = jax.lax.optimization_barrier(cb_bf[:, :K_SC]).astype(_f32)
    cb_tc = cb_bf[:, K_SC:]
    b1r = b1.reshape(1, H)
    b2r = b2.reshape(1, H)
    b3r = b3.reshape(1, LAT)
    pbr = proj_b.reshape(1, CD)
    zn = pl.pallas_call(
        _enc_body,
        grid=(B,),
        in_specs=[
            pl.BlockSpec((1, T, D), lambda b: (b, 0, 0)),
            pl.BlockSpec((3, D, H), lambda b: (0, 0, 0)),
            pl.BlockSpec((1, H), lambda b: (0, 0)),
            pl.BlockSpec((3, H, H), lambda b: (0, 0, 0)),
            pl.BlockSpec((1, H), lambda b: (0, 0)),
            pl.BlockSpec((H, LAT), lambda b: (0, 0)),
            pl.BlockSpec((1, LAT), lambda b: (0, 0)),
            pl.BlockSpec((LAT, CD), lambda b: (0, 0)),
            pl.BlockSpec((1, CD), lambda b: (0, 0)),
        ],
        out_specs=pl.BlockSpec((1, T, CD), lambda b: (b, 0, 0)),
        out_shape=jax.ShapeDtypeStruct((B, T, CD), _bf16),
    )(x_bf, w1k, b1r, w2k, b2r, w3t, b3r, pwt, pbr)
    zt = jnp.transpose(zn.reshape(B * T, CD), (1, 0)).astype(_f32)
    sc_v, sc_i = pl.kernel(
        _tok_body,
        out_type=[
            jax.ShapeDtypeStruct((B * T,), _f32),
            jax.ShapeDtypeStruct((B * T,), jnp.int32),
        ],
        mesh=plsc.VectorSubcoreMesh(core_axis_name="c", subcore_axis_name="s"),
        scratch_types=[
            pltpu.VMEM((CD, K_SC), _f32),
            pltpu.VMEM((CD, _TPW), _f32),
            pltpu.VMEM((_TPW,), _f32),
            pltpu.VMEM((_TPW,), jnp.int32),
        ],
    )(zt, cb_sc)
    tc_v, tc_i = pl.pallas_call(
        _scan_tc_body,
        grid=(B,),
        in_specs=[
            pl.BlockSpec((1, T, CD), lambda b: (b, 0, 0)),
            pl.BlockSpec((CD, K - K_SC), lambda b: (0, 0)),
        ],
        out_specs=[
            pl.BlockSpec((1, T, 1), lambda b: (b, 0, 0)),
            pl.BlockSpec((1, T, 1), lambda b: (b, 0, 0)),
        ],
        out_shape=[
            jax.ShapeDtypeStruct((B, T, 1), _f32),
            jax.ShapeDtypeStruct((B, T, 1), jnp.int32),
        ],
    )(zn, cb_tc)
    tc_vf = tc_v.reshape(B * T)
    tc_if = tc_i.reshape(B * T)
    tok = jnp.where(sc_v >= tc_vf, sc_i, tc_if)
    return tok.reshape(B, T)


# speedup vs baseline: 2.7895x; 1.0791x over previous
import functools

import jax
import jax.numpy as jnp
from jax.experimental import pallas as pl
from jax.experimental.pallas import tpu as pltpu
from jax.experimental.pallas import tpu_sc as plsc

B, T, D = 4, 1024, 1024
H = 512
LAT = 1024
K = 8192
CD = 8

K_SC = 512
KC = 512

_f32 = jnp.float32
_bf16 = jnp.bfloat16

_NW = 32
_TPW = (B * T) // _NW
_NGRP = _TPW // 16
_GPP = 4


def _bdot(a_bf, b_bf):
    return jnp.dot(a_bf, b_bf, preferred_element_type=_f32)


def _enc_body(x_ref, w1_ref, b1_ref, w2_ref, b2_ref, w3_ref, b3_ref,
              pw_ref, pb_ref, zn_ref):
    x = x_ref[0]

    def conv3tap(inp_bf, w_ref, b_row, width):
        y0 = _bdot(inp_bf, w_ref[0])
        y1 = _bdot(inp_bf, w_ref[1])
        y2 = _bdot(inp_bf, w_ref[2])
        zrow = jnp.zeros((1, width), _f32)
        return (jnp.concatenate([zrow, y0[:-1]], axis=0) + y1
                + jnp.concatenate([y2[1:], zrow], axis=0) + b_row)

    h1 = jax.nn.gelu(conv3tap(x, w1_ref, b1_ref[...], H))
    h2 = jax.nn.gelu(conv3tap(h1.astype(_bf16), w2_ref, b2_ref[...], H) + h1)
    z = _bdot(h2.astype(_bf16), w3_ref[...]) + b3_ref[...]
    zp = _bdot(z.astype(_bf16), pw_ref[...]) + pb_ref[...]
    zn = zp / (jnp.sqrt(jnp.sum(zp * zp, axis=1, keepdims=True)) + 1e-8)
    zn_ref[0] = zn.astype(_bf16)


def _scan_tc_body(zn_ref, cbn_ref, bv_ref, bi_ref):
    zn_bf = zn_ref[0]
    best_v = jnp.full((T, 1), -jnp.inf, _f32)
    best_i = jnp.zeros((T, 1), jnp.int32)
    for kc in range(0, K - K_SC, KC):
        s = _bdot(zn_bf, cbn_ref[:, kc:kc + KC])
        m = jnp.max(s, axis=1, keepdims=True)
        idx = jax.lax.broadcasted_iota(jnp.int32, s.shape, 1) + (K_SC + kc)
        cand = jnp.min(jnp.where(s == m, idx, K), axis=1, keepdims=True)
        upd = m > best_v
        best_v = jnp.where(upd, m, best_v)
        best_i = jnp.where(upd, cand, best_i)
    bv_ref[0] = best_v
    bi_ref[0] = best_i


def _tok_body(zt_hbm, cb_hbm, bv_hbm, bi_hbm, cb_v, z_v, val_v, idx_v):
    cid = jax.lax.axis_index("c")
    sid = jax.lax.axis_index("s")
    wid = sid * 2 + cid
    base = wid * _TPW
    pltpu.sync_copy(cb_hbm, cb_v)
    pltpu.sync_copy(zt_hbm.at[:, pl.ds(base, _TPW)], z_v)
    ninf = jnp.full((16,), -jnp.inf, _f32)
    zero = jnp.zeros((16,), jnp.int32)
    for g0 in range(0, _NGRP, _GPP):
        zs = [[z_v[c, pl.ds((g0 + j) * 16, 16)] for c in range(CD)]
              for j in range(_GPP)]

        def body(k0, carry, zs=zs):
            kbase = k0 * 16
            cvecs = [cb_v[c, pl.ds(kbase, 16)] for c in range(CD)]
            kvec0 = jnp.full((16,), kbase, jnp.int32)
            out = list(carry)
            for t in range(16):
                s = [cvecs[c][t] for c in range(CD)]
                kvec = kvec0 + t
                for j in range(_GPP):
                    bv, bi = out[2 * j], out[2 * j + 1]
                    sim = zs[j][0] * s[0]
                    for c in range(1, CD):
                        sim = sim + zs[j][c] * s[c]
                    m = sim > bv
                    out[2 * j] = jnp.where(m, sim, bv)
                    out[2 * j + 1] = jnp.where(m, kvec, bi)
            return tuple(out)

        carry = jax.lax.fori_loop(0, K_SC // 16, body, (ninf, zero) * _GPP)
        for j in range(_GPP):
            val_v[pl.ds((g0 + j) * 16, 16)] = carry[2 * j]
            idx_v[pl.ds((g0 + j) * 16, 16)] = carry[2 * j + 1]
    pltpu.sync_copy(val_v, bv_hbm.at[pl.ds(base, _TPW)])
    pltpu.sync_copy(idx_v, bi_hbm.at[pl.ds(base, _TPW)])


def kernel(features, w1, b1, w2, b2, w3, b3, proj_w, proj_b, codebook):
    x_bf = features.astype(_bf16)
    w1k = jnp.transpose(w1, (2, 1, 0)).astype(_bf16)
    w2k = jnp.transpose(w2, (2, 1, 0)).astype(_bf16)
    w3t = jnp.transpose(w3[:, :, 0], (1, 0)).astype(_bf16)
    pwt = jnp.transpose(proj_w, (1, 0)).astype(_bf16)
    cbn = codebook / (jnp.linalg.norm(codebook, axis=-1, keepdims=True) + 1e-8)
    cb_bf = jnp.transpose(cbn, (1, 0)).astype(_bf16)
    cb_sc = jax.lax.optimization_barrier(cb_bf[:, :K_SC]).astype(_f32)
    cb_tc = cb_bf[:, K_SC:]
    b1r = b1.reshape(1, H)
    b2r = b2.reshape(1, H)
    b3r = b3.reshape(1, LAT)
    pbr = proj_b.reshape(1, CD)
    zn = pl.pallas_call(
        _enc_body,
        grid=(B,),
        in_specs=[
            pl.BlockSpec((1, T, D), lambda b: (b, 0, 0)),
            pl.BlockSpec((3, D, H), lambda b: (0, 0, 0)),
            pl.BlockSpec((1, H), lambda b: (0, 0)),
            pl.BlockSpec((3, H, H), lambda b: (0, 0, 0)),
            pl.BlockSpec((1, H), lambda b: (0, 0)),
            pl.BlockSpec((H, LAT), lambda b: (0, 0)),
            pl.BlockSpec((1, LAT), lambda b: (0, 0)),
            pl.BlockSpec((LAT, CD), lambda b: (0, 0)),
            pl.BlockSpec((1, CD), lambda b: (0, 0)),
        ],
        out_specs=pl.BlockSpec((1, T, CD), lambda b: (b, 0, 0)),
        out_shape=jax.ShapeDtypeStruct((B, T, CD), _bf16),
    )(x_bf, w1k, b1r, w2k, b2r, w3t, b3r, pwt, pbr)
    zt = jnp.transpose(zn.reshape(B * T, CD), (1, 0)).astype(_f32)
    sc_v, sc_i = pl.kernel(
        _tok_body,
        out_type=[
            jax.ShapeDtypeStruct((B * T,), _f32),
            jax.ShapeDtypeStruct((B * T,), jnp.int32),
        ],
        mesh=plsc.VectorSubcoreMesh(core_axis_name="c", subcore_axis_name="s"),
        scratch_types=[
            pltpu.VMEM((CD, K_SC), _f32),
            pltpu.VMEM((CD, _TPW), _f32),
            pltpu.VMEM((_TPW,), _f32),
            pltpu.VMEM((_TPW,), jnp.int32),
        ],
    )(zt, cb_sc)
    tc_v, tc_i = pl.pallas_call(
        _scan_tc_body,
        grid=(B,),
        in_specs=[
            pl.BlockSpec((1, T, CD), lambda b: (b, 0, 0)),
            pl.BlockSpec((CD, K - K_SC), lambda b: (0, 0)),
        ],
        out_specs=[
            pl.BlockSpec((1, T, 1), lambda b: (b, 0, 0)),
            pl.BlockSpec((1, T, 1), lambda b: (b, 0, 0)),
        ],
        out_shape=[
            jax.ShapeDtypeStruct((B, T, 1), _f32),
            jax.ShapeDtypeStruct((B, T, 1), jnp.int32),
        ],
    )(zn, cb_tc)
    tc_vf = tc_v.reshape(B * T)
    tc_if = tc_i.reshape(B * T)
    tok = jnp.where(sc_v >= tc_vf, sc_i, tc_if)
    return tok.reshape(B, T)
